# Initial kernel scaffold; baseline (speedup 1.0000x reference)
import jax
import jax.numpy as jnp
from jax.experimental import pallas as pl


def kernel(x, edge_index, W_enc, b_enc, W1, b1, g1, beta1, W2, b2, g2, beta2, W_sr, b_sr):
    raise NotImplementedError("write your pallas kernel here")


# speedup vs baseline: 9.6875x; 9.6875x over previous
import functools

import jax
import jax.numpy as jnp
from jax import lax
from jax.experimental import pallas as pl
from jax.experimental.pallas import tpu as pltpu
from jax.experimental.pallas import tpu_sc as plsc

EPS = 1e-5
_NC = 2
_NS = 16
_K = 128
_DEGW = 8


def _make_sc_degree(e_pad, n_half, rpt):
    g_cnt = e_pad // (_NS * _K)
    mesh = plsc.VectorSubcoreMesh(core_axis_name="c", subcore_axis_name="s")

    @functools.partial(
        pl.kernel,
        out_type=jax.ShapeDtypeStruct((_NC * n_half, _DEGW), jnp.float32),
        mesh=mesh,
        compiler_params=pltpu.CompilerParams(use_tc_tiling_on_sc=False),
        scratch_types=[
            pltpu.VMEM_SHARED((n_half + 8, _DEGW), jnp.float32),
            pltpu.VMEM((rpt, _DEGW), jnp.float32),
            pltpu.VMEM((_K,), jnp.int32),
            pltpu.VMEM((_K,), jnp.int32),
            pltpu.VMEM((_K, _DEGW), jnp.float32),
        ],
    )
    def deg_kernel(dst_hbm, ones_hbm, zeros_hbm, out_hbm,
                   acc, stage, dst_v, idx_v, ones_v):
        c = lax.axis_index("c")
        s = lax.axis_index("s")
        base = c * n_half
        pltpu.sync_copy(zeros_hbm, stage)
        pltpu.sync_copy(ones_hbm, ones_v)
        pltpu.sync_copy(stage, acc.at[pl.ds(s * rpt, rpt)])
        plsc.subcore_barrier()

        def body(g, carry):
            off = (s * g_cnt + g) * _K
            pltpu.sync_copy(dst_hbm.at[pl.ds(off, _K)], dst_v)
            for i in range(_K // 16):
                d = dst_v[pl.ds(i * 16, 16)]
                loc = d - base
                ok = (loc >= 0) & (loc < n_half)
                idx_v[pl.ds(i * 16, 16)] = jnp.where(ok, loc, n_half)
            pltpu.sync_copy(ones_v, acc.at[idx_v], add=True)
            return carry

        lax.fori_loop(0, g_cnt, body, 0)
        plsc.subcore_barrier()
        pltpu.sync_copy(acc.at[pl.ds(s * rpt, rpt)], stage)
        pltpu.sync_copy(stage, out_hbm.at[pl.ds(base + s * rpt, rpt)])

    return deg_kernel


def _make_sc_aggregate(e_pad, n_half, rpt, h_dim):
    g_cnt = e_pad // (_NS * _K)
    wb = rpt // 14
    n_wb = rpt // wb
    mesh = plsc.VectorSubcoreMesh(core_axis_name="c", subcore_axis_name="s")

    @functools.partial(
        pl.kernel,
        out_type=jax.ShapeDtypeStruct((_NC * n_half, h_dim), jnp.float32),
        mesh=mesh,
        compiler_params=pltpu.CompilerParams(use_tc_tiling_on_sc=False),
        scratch_types=[
            pltpu.VMEM_SHARED((n_half + 8, h_dim), jnp.float32),
            pltpu.VMEM((wb, h_dim), jnp.float32),
            pltpu.VMEM((_K,), jnp.int32),
            pltpu.VMEM((_K,), jnp.int32),
            pltpu.VMEM((_K,), jnp.int32),
            pltpu.VMEM((_K, h_dim), jnp.float32),
            pltpu.SemaphoreType.DMA,
        ],
    )
    def agg_kernel(rows_hbm, src_hbm, dst_hbm, zeros_hbm, out_hbm,
                   acc, stage, src_v, dst_v, idx_v, rows_v, sem):
        c = lax.axis_index("c")
        s = lax.axis_index("s")
        base = c * n_half
        pltpu.sync_copy(zeros_hbm, stage)
        for k in range(n_wb):
            pltpu.sync_copy(stage, acc.at[pl.ds(s * rpt + k * wb, wb)])
        plsc.subcore_barrier()

        def body(g, carry):
            off = (s * g_cnt + g) * _K
            pltpu.sync_copy(src_hbm.at[pl.ds(off, _K)], src_v)
            pltpu.sync_copy(dst_hbm.at[pl.ds(off, _K)], dst_v)
            for i in range(_K // 16):
                d = dst_v[pl.ds(i * 16, 16)]
                loc = d - base
                ok = (loc >= 0) & (loc < n_half)
                idx_v[pl.ds(i * 16, 16)] = jnp.where(ok, loc, n_half)
            pltpu.async_copy(rows_hbm.at[src_v], rows_v, sem).wait()
            pltpu.sync_copy(rows_v, acc.at[idx_v], add=True)
            return carry

        lax.fori_loop(0, g_cnt, body, 0)
        plsc.subcore_barrier()
        for k in range(n_wb):
            off_loc = s * rpt + k * wb
            pltpu.sync_copy(acc.at[pl.ds(off_loc, wb)], stage)
            pltpu.sync_copy(stage, out_hbm.at[pl.ds(base + off_loc, wb)])

    return agg_kernel


def _prep_body(x_ref, deg_ref, we_ref, be_ref, w1_ref, hw_ref, dis_ref):
    h0 = jnp.maximum(
        jnp.dot(x_ref[...], we_ref[...], preferred_element_type=jnp.float32)
        + be_ref[...], 0.0)
    deg = deg_ref[:, 0:1] + 1.0
    dis = lax.rsqrt(jnp.maximum(deg, 1.0))
    hw = jnp.dot(h0, w1_ref[...], preferred_element_type=jnp.float32)
    hw_ref[...] = hw * dis
    dis_ref[...] = dis


def _mid_body(s_ref, hw_ref, dis_ref, b_ref, g_ref, bet_ref, w_ref, out_ref):
    dis = dis_ref[...]
    z = dis * (s_ref[...] + hw_ref[...]) + b_ref[...]
    mu = jnp.mean(z, axis=-1, keepdims=True)
    zc = z - mu
    var = jnp.mean(zc * zc, axis=-1, keepdims=True)
    h = jnp.maximum(zc * lax.rsqrt(var + EPS) * g_ref[...] + bet_ref[...], 0.0)
    out_ref[...] = jnp.dot(h, w_ref[...], preferred_element_type=jnp.float32) * dis


def _final_body(s_ref, hw_ref, dis_ref, b_ref, g_ref, bet_ref, w_ref, bsr_ref,
                out_ref):
    dis = dis_ref[...]
    z = dis * (s_ref[...] + hw_ref[...]) + b_ref[...]
    mu = jnp.mean(z, axis=-1, keepdims=True)
    zc = z - mu
    var = jnp.mean(zc * zc, axis=-1, keepdims=True)
    h = jnp.maximum(zc * lax.rsqrt(var + EPS) * g_ref[...] + bet_ref[...], 0.0)
    out_ref[...] = jnp.tanh(
        jnp.dot(h, w_ref[...], preferred_element_type=jnp.float32) + bsr_ref[...])


def _row_block(n, blk, d):
    return pl.BlockSpec((blk, d), lambda i: (i, 0))


def _whole(shape):
    return pl.BlockSpec(shape, lambda i: (0, 0))


def kernel(x, edge_index, W_enc, b_enc, W1, b1, g1, beta1, W2, b2, g2, beta2,
           W_sr, b_sr):
    n, d = x.shape
    e = edge_index.shape[1]
    h_dim = W1.shape[0]

    rpt = -(-n // (_NC * _NS * 32)) * 32
    n_half = _NS * rpt
    ept = -(-e // (_NS * _K)) * _K
    e_pad = _NS * ept
    pad = e_pad - e
    src_p = jnp.concatenate([edge_index[0], jnp.zeros((pad,), jnp.int32)])
    dst_p = jnp.concatenate(
        [edge_index[1], jnp.full((pad,), jnp.int32(1 << 20))])

    ones_deg = jnp.ones((_K, _DEGW), jnp.float32)
    zeros_deg = jnp.zeros((rpt, _DEGW), jnp.float32)
    zeros_agg = jnp.zeros((rpt // 14, h_dim), jnp.float32)

    deg8 = _make_sc_degree(e_pad, n_half, rpt)(dst_p, ones_deg, zeros_deg)

    blk = 2000
    grid = (n // blk,)
    hw1p, dis = pl.pallas_call(
        _prep_body,
        grid=grid,
        in_specs=[
            _row_block(n, blk, d),
            _row_block(n, blk, _DEGW),
            _whole((d, h_dim)),
            _whole((1, h_dim)),
            _whole((h_dim, h_dim)),
        ],
        out_specs=[_row_block(n, blk, h_dim), _row_block(n, blk, 1)],
        out_shape=[
            jax.ShapeDtypeStruct((n, h_dim), jnp.float32),
            jax.ShapeDtypeStruct((n, 1), jnp.float32),
        ],
    )(x, deg8[:n], W_enc, b_enc.reshape(1, -1), W1)

    agg = _make_sc_aggregate(e_pad, n_half, rpt, h_dim)

    s1 = agg(hw1p, src_p, dst_p, zeros_agg)
    hw2p = pl.pallas_call(
        _mid_body,
        grid=grid,
        in_specs=[
            _row_block(n, blk, h_dim),
            _row_block(n, blk, h_dim),
            _row_block(n, blk, 1),
            _whole((1, h_dim)),
            _whole((1, h_dim)),
            _whole((1, h_dim)),
            _whole((h_dim, h_dim)),
        ],
        out_specs=_row_block(n, blk, h_dim),
        out_shape=jax.ShapeDtypeStruct((n, h_dim), jnp.float32),
    )(s1[:n], hw1p, dis, b1.reshape(1, -1), g1.reshape(1, -1),
      beta1.reshape(1, -1), W2)

    s2 = agg(hw2p, src_p, dst_p, zeros_agg)
    belief = pl.pallas_call(
        _final_body,
        grid=grid,
        in_specs=[
            _row_block(n, blk, h_dim),
            _row_block(n, blk, h_dim),
            _row_block(n, blk, 1),
            _whole((1, h_dim)),
            _whole((1, h_dim)),
            _whole((1, h_dim)),
            _whole((h_dim, h_dim)),
            _whole((1, h_dim)),
        ],
        out_specs=_row_block(n, blk, h_dim),
        out_shape=jax.ShapeDtypeStruct((n, h_dim), jnp.float32),
    )(s2[:n], hw2p, dis, b2.reshape(1, -1), g2.reshape(1, -1),
      beta2.reshape(1, -1), W_sr, b_sr.reshape(1, -1))

    return belief


# speedup vs baseline: 12.4165x; 1.2817x over previous
import functools

import jax
import jax.numpy as jnp
from jax import lax
from jax.experimental import pallas as pl
from jax.experimental.pallas import tpu as pltpu
from jax.experimental.pallas import tpu_sc as plsc

EPS = 1e-5
_NC = 2
_NS = 16
_K = 128
_DEGW = 8


def _idx_from_dst(ebuf, idx, base, n_half):
    for i in range(_K // 16):
        d = ebuf[1, pl.ds(i * 16, 16)]
        loc = d - base
        ok = (loc >= 0) & (loc < n_half)
        idx[pl.ds(i * 16, 16)] = jnp.where(ok, loc, n_half)


def _make_sc_degree(e_pad, n_half, rpt):
    g_cnt = e_pad // (_NS * _K)
    assert g_cnt % 2 == 0
    mesh = plsc.VectorSubcoreMesh(core_axis_name="c", subcore_axis_name="s")

    @functools.partial(
        pl.kernel,
        out_type=jax.ShapeDtypeStruct((_NC * n_half, _DEGW), jnp.float32),
        mesh=mesh,
        compiler_params=pltpu.CompilerParams(use_tc_tiling_on_sc=False),
        scratch_types=[
            pltpu.VMEM_SHARED((n_half + 8, _DEGW), jnp.float32),
            pltpu.VMEM((rpt, _DEGW), jnp.float32),
            [pltpu.VMEM((2, _K), jnp.int32)] * 2,
            [pltpu.VMEM((_K,), jnp.int32)] * 2,
            pltpu.VMEM((_K, _DEGW), jnp.float32),
            [pltpu.SemaphoreType.DMA] * 2,
            [pltpu.SemaphoreType.DMA] * 2,
        ],
    )
    def deg_kernel(edges_hbm, ones_hbm, zeros_hbm, out_hbm,
                   acc, stage, ebufs, idxs, ones_v, esems, ssems):
        c = lax.axis_index("c")
        s = lax.axis_index("s")
        base = c * n_half
        pltpu.sync_copy(zeros_hbm, stage)
        pltpu.sync_copy(ones_hbm, ones_v)
        pltpu.sync_copy(stage, acc.at[pl.ds(s * rpt, rpt)])
        plsc.subcore_barrier()

        for b in range(2):
            pltpu.async_copy(edges_hbm.at[s * g_cnt + b], ebufs[b], esems[b])

        def pair(t, carry):
            for b in range(2):
                g = 2 * t + b

                @pl.when(t > 0)
                def _wait_prev():
                    pltpu.make_async_copy(
                        ones_v, acc.at[idxs[b]], ssems[b]).wait()

                pltpu.make_async_copy(
                    edges_hbm.at[0], ebufs[b], esems[b]).wait()
                _idx_from_dst(ebufs[b], idxs[b], base, n_half)

                @pl.when(g + 2 < g_cnt)
                def _prefetch():
                    pltpu.async_copy(
                        edges_hbm.at[s * g_cnt + g + 2], ebufs[b], esems[b])

                pltpu.async_copy(ones_v, acc.at[idxs[b]], ssems[b], add=True)
            return carry

        lax.fori_loop(0, g_cnt // 2, pair, 0)
        for b in range(2):
            pltpu.make_async_copy(ones_v, acc.at[idxs[b]], ssems[b]).wait()
        plsc.subcore_barrier()
        pltpu.sync_copy(acc.at[pl.ds(s * rpt, rpt)], stage)
        pltpu.sync_copy(stage, out_hbm.at[pl.ds(base + s * rpt, rpt)])

    return deg_kernel


def _make_sc_aggregate(e_pad, n_half, rpt, h_dim):
    g_cnt = e_pad // (_NS * _K)
    wb = rpt // 14
    n_wb = rpt // wb
    mesh = plsc.VectorSubcoreMesh(core_axis_name="c", subcore_axis_name="s")

    assert g_cnt % 2 == 0

    @functools.partial(
        pl.kernel,
        out_type=jax.ShapeDtypeStruct((_NC * n_half, h_dim), jnp.float32),
        mesh=mesh,
        compiler_params=pltpu.CompilerParams(use_tc_tiling_on_sc=False),
        scratch_types=[
            pltpu.VMEM_SHARED((n_half + 8, h_dim), jnp.float32),
            pltpu.VMEM((wb, h_dim), jnp.float32),
            [pltpu.VMEM((2, _K), jnp.int32)] * 2,
            [pltpu.VMEM((_K,), jnp.int32)] * 2,
            [pltpu.VMEM((_K, h_dim), jnp.float32)] * 2,
            [pltpu.SemaphoreType.DMA] * 2,
            pltpu.SemaphoreType.DMA,
            [pltpu.SemaphoreType.DMA] * 2,
        ],
    )
    def agg_kernel(rows_hbm, edges_hbm, zeros_hbm, out_hbm,
                   acc, stage, ebufs, idxs, rows, esems, gsem, ssems):
        c = lax.axis_index("c")
        s = lax.axis_index("s")
        base = c * n_half
        pltpu.sync_copy(zeros_hbm, stage)
        for k in range(n_wb):
            pltpu.sync_copy(stage, acc.at[pl.ds(s * rpt + k * wb, wb)])
        plsc.subcore_barrier()

        for b in range(2):
            pltpu.async_copy(edges_hbm.at[s * g_cnt + b], ebufs[b], esems[b])

        def pair(t, carry):
            for b in range(2):
                g = 2 * t + b

                @pl.when(t > 0)
                def _wait_prev():
                    pltpu.make_async_copy(
                        rows[b], acc.at[idxs[b]], ssems[b]).wait()

                pltpu.make_async_copy(
                    edges_hbm.at[0], ebufs[b], esems[b]).wait()
                _idx_from_dst(ebufs[b], idxs[b], base, n_half)
                pltpu.async_copy(
                    rows_hbm.at[ebufs[b].at[0]], rows[b], gsem).wait()

                @pl.when(g + 2 < g_cnt)
                def _prefetch():
                    pltpu.async_copy(
                        edges_hbm.at[s * g_cnt + g + 2], ebufs[b], esems[b])

                pltpu.async_copy(rows[b], acc.at[idxs[b]], ssems[b], add=True)
            return carry

        lax.fori_loop(0, g_cnt // 2, pair, 0)
        for b in range(2):
            pltpu.make_async_copy(rows[b], acc.at[idxs[b]], ssems[b]).wait()
        plsc.subcore_barrier()
        for k in range(n_wb):
            off_loc = s * rpt + k * wb
            pltpu.sync_copy(acc.at[pl.ds(off_loc, wb)], stage)
            pltpu.sync_copy(stage, out_hbm.at[pl.ds(base + off_loc, wb)])

    return agg_kernel


def _prep_body(x_ref, deg_ref, we_ref, be_ref, w1_ref, hw_ref, dis_ref):
    h0 = jnp.maximum(
        jnp.dot(x_ref[...], we_ref[...], preferred_element_type=jnp.float32)
        + be_ref[...], 0.0)
    deg = deg_ref[:, 0:1] + 1.0
    dis = lax.rsqrt(jnp.maximum(deg, 1.0))
    hw = jnp.dot(h0, w1_ref[...], preferred_element_type=jnp.float32)
    hw_ref[...] = hw * dis
    dis_ref[...] = dis


def _mid_body(s_ref, hw_ref, dis_ref, b_ref, g_ref, bet_ref, w_ref, out_ref):
    dis = dis_ref[...]
    z = dis * (s_ref[...] + hw_ref[...]) + b_ref[...]
    mu = jnp.mean(z, axis=-1, keepdims=True)
    zc = z - mu
    var = jnp.mean(zc * zc, axis=-1, keepdims=True)
    h = jnp.maximum(zc * lax.rsqrt(var + EPS) * g_ref[...] + bet_ref[...], 0.0)
    out_ref[...] = jnp.dot(h, w_ref[...], preferred_element_type=jnp.float32) * dis


def _final_body(s_ref, hw_ref, dis_ref, b_ref, g_ref, bet_ref, w_ref, bsr_ref,
                out_ref):
    dis = dis_ref[...]
    z = dis * (s_ref[...] + hw_ref[...]) + b_ref[...]
    mu = jnp.mean(z, axis=-1, keepdims=True)
    zc = z - mu
    var = jnp.mean(zc * zc, axis=-1, keepdims=True)
    h = jnp.maximum(zc * lax.rsqrt(var + EPS) * g_ref[...] + bet_ref[...], 0.0)
    out_ref[...] = jnp.tanh(
        jnp.dot(h, w_ref[...], preferred_element_type=jnp.float32) + bsr_ref[...])


def _row_block(n, blk, d):
    return pl.BlockSpec((blk, d), lambda i: (i, 0))


def _whole(shape):
    return pl.BlockSpec(shape, lambda i: (0, 0))


def kernel(x, edge_index, W_enc, b_enc, W1, b1, g1, beta1, W2, b2, g2, beta2,
           W_sr, b_sr):
    n, d = x.shape
    e = edge_index.shape[1]
    h_dim = W1.shape[0]

    rpt = -(-n // (_NC * _NS * 32)) * 32
    n_half = _NS * rpt
    ept = -(-e // (_NS * 2 * _K)) * 2 * _K
    e_pad = _NS * ept
    pad = e_pad - e
    src_p = jnp.concatenate([edge_index[0], jnp.zeros((pad,), jnp.int32)])
    dst_p = jnp.concatenate(
        [edge_index[1], jnp.full((pad,), jnp.int32(1 << 20))])
    edges_p = jnp.stack(
        [src_p.reshape(-1, _K), dst_p.reshape(-1, _K)], axis=1)

    ones_deg = jnp.ones((_K, _DEGW), jnp.float32)
    zeros_deg = jnp.zeros((rpt, _DEGW), jnp.float32)
    zeros_agg = jnp.zeros((rpt // 14, h_dim), jnp.float32)

    deg8 = _make_sc_degree(e_pad, n_half, rpt)(edges_p, ones_deg, zeros_deg)

    blk = 2000
    grid = (n // blk,)
    hw1p, dis = pl.pallas_call(
        _prep_body,
        grid=grid,
        in_specs=[
            _row_block(n, blk, d),
            _row_block(n, blk, _DEGW),
            _whole((d, h_dim)),
            _whole((1, h_dim)),
            _whole((h_dim, h_dim)),
        ],
        out_specs=[_row_block(n, blk, h_dim), _row_block(n, blk, 1)],
        out_shape=[
            jax.ShapeDtypeStruct((n, h_dim), jnp.float32),
            jax.ShapeDtypeStruct((n, 1), jnp.float32),
        ],
    )(x, deg8[:n], W_enc, b_enc.reshape(1, -1), W1)

    agg = _make_sc_aggregate(e_pad, n_half, rpt, h_dim)

    s1 = agg(hw1p, edges_p, zeros_agg)
    hw2p = pl.pallas_call(
        _mid_body,
        grid=grid,
        in_specs=[
            _row_block(n, blk, h_dim),
            _row_block(n, blk, h_dim),
            _row_block(n, blk, 1),
            _whole((1, h_dim)),
            _whole((1, h_dim)),
            _whole((1, h_dim)),
            _whole((h_dim, h_dim)),
        ],
        out_specs=_row_block(n, blk, h_dim),
        out_shape=jax.ShapeDtypeStruct((n, h_dim), jnp.float32),
    )(s1[:n], hw1p, dis, b1.reshape(1, -1), g1.reshape(1, -1),
      beta1.reshape(1, -1), W2)

    s2 = agg(hw2p, edges_p, zeros_agg)
    belief = pl.pallas_call(
        _final_body,
        grid=grid,
        in_specs=[
            _row_block(n, blk, h_dim),
            _row_block(n, blk, h_dim),
            _row_block(n, blk, 1),
            _whole((1, h_dim)),
            _whole((1, h_dim)),
            _whole((1, h_dim)),
            _whole((h_dim, h_dim)),
            _whole((1, h_dim)),
        ],
        out_specs=_row_block(n, blk, h_dim),
        out_shape=jax.ShapeDtypeStruct((n, h_dim), jnp.float32),
    )(s2[:n], hw2p, dis, b2.reshape(1, -1), g2.reshape(1, -1),
      beta2.reshape(1, -1), W_sr, b_sr.reshape(1, -1))

    return belief
